# Initial kernel scaffold; baseline (speedup 1.0000x reference)
import jax
import jax.numpy as jnp
from jax.experimental import pallas as pl


def kernel(x, edge_index, Wl0, bl0, Wr0, br0, Wl1, bl1, Wr1, br1, Wl2, bl2, Wr2, br2, Wout, bout):
    raise NotImplementedError("write your pallas kernel here")


# speedup vs baseline: 3.0420x; 3.0420x over previous
import functools

import jax
import jax.numpy as jnp
from jax import lax
from jax.experimental import pallas as pl
from jax.experimental.pallas import tpu as pltpu
from jax.experimental.pallas import tpu_sc as plsc

N = 10000
E = 320000
D = 128
NC, NS = 2, 16
NT = NC * NS
CHUNK = 128
EPT = 10240
EPAD = NT * EPT
NITER = EPT // CHUNK
NPAD = 10240
RPT = NPAD // NS
OPT = 624
HR = NPAD // D
BR = 1000


def _sc_segsum_body(h, srcp, dstp, zrows, out0, out1,
                    acc, idx_v, didx_v, rows_v, sem):
    c = lax.axis_index("c")
    s = lax.axis_index("s")
    tile = c * NS + s

    pltpu.sync_copy(zrows, acc.at[pl.ds(s * RPT, RPT)])
    plsc.subcore_barrier()

    def step(i, carry):
        eoff = tile * EPT + i * CHUNK
        pltpu.sync_copy(srcp.at[pl.ds(eoff, CHUNK)], idx_v)
        pltpu.sync_copy(dstp.at[pl.ds(eoff, CHUNK)], didx_v)
        pltpu.async_copy(h.at[idx_v], rows_v, sem).wait()
        pltpu.sync_copy(rows_v, acc.at[didx_v], add=True)
        return carry

    lax.fori_loop(0, NITER, step, 0)
    plsc.subcore_barrier()

    rows = pl.ds(s * OPT, OPT)
    tail = pl.ds(NS * OPT, N - NS * OPT)

    @pl.when(c == 0)
    def _():
        pltpu.sync_copy(acc.at[rows], out0.at[rows])

    @pl.when(c == 1)
    def _():
        pltpu.sync_copy(acc.at[rows], out1.at[rows])

    @pl.when((c == 0) & (s == NS - 1))
    def _():
        pltpu.sync_copy(acc.at[tail], out0.at[tail])

    @pl.when((c == 1) & (s == NS - 1))
    def _():
        pltpu.sync_copy(acc.at[tail], out1.at[tail])


_sc_segsum = pl.kernel(
    _sc_segsum_body,
    out_type=(jax.ShapeDtypeStruct((N, D), jnp.float32),
              jax.ShapeDtypeStruct((N, D), jnp.float32)),
    mesh=plsc.VectorSubcoreMesh(core_axis_name="c", subcore_axis_name="s"),
    scratch_types=[
        pltpu.VMEM_SHARED((NPAD, D), jnp.float32),
        pltpu.VMEM((CHUNK,), jnp.int32),
        pltpu.VMEM((CHUNK,), jnp.int32),
        pltpu.VMEM((CHUNK, D), jnp.float32),
        pltpu.SemaphoreType.DMA,
    ],
)


def _sc_degree_body(dstp, zrows, iota_hbm, out0, out1,
                    acc, hist, didx_v, idx80, sem):
    c = lax.axis_index("c")
    s = lax.axis_index("s")
    tile = c * NS + s

    @pl.when(s < HR // 8)
    def _():
        pltpu.sync_copy(zrows.at[pl.ds(0, 8)], acc.at[pl.ds(s * 8, 8)])

    pltpu.sync_copy(zrows.at[pl.ds(0, HR)], hist)
    pltpu.sync_copy(iota_hbm, idx80)
    plsc.subcore_barrier()

    def step(i, carry):
        eoff = tile * EPT + i * CHUNK
        pltpu.sync_copy(dstp.at[pl.ds(eoff, CHUNK)], didx_v)
        for k in range(CHUNK // 16):
            d16 = didx_v[pl.ds(k * 16, 16)]
            cnt, last = plsc.scan_count(d16)
            plsc.addupdate_scatter(
                hist,
                [lax.shift_right_logical(d16, 7),
                 lax.bitwise_and(d16, 127)],
                cnt.astype(jnp.float32),
                mask=last,
            )
        return carry

    lax.fori_loop(0, NITER, step, 0)
    pltpu.sync_copy(hist, acc.at[idx80], add=True)
    plsc.subcore_barrier()

    @pl.when((c == 0) & (s < HR // 8))
    def _():
        pltpu.sync_copy(acc.at[pl.ds(s * 8, 8)], out0.at[pl.ds(s * 8, 8)])

    @pl.when((c == 1) & (s < HR // 8))
    def _():
        pltpu.sync_copy(acc.at[pl.ds(s * 8, 8)], out1.at[pl.ds(s * 8, 8)])


_sc_degree = pl.kernel(
    _sc_degree_body,
    out_type=(jax.ShapeDtypeStruct((HR, D), jnp.float32),
              jax.ShapeDtypeStruct((HR, D), jnp.float32)),
    mesh=plsc.VectorSubcoreMesh(core_axis_name="c", subcore_axis_name="s"),
    scratch_types=[
        pltpu.VMEM_SHARED((HR, D), jnp.float32),
        pltpu.VMEM((HR, D), jnp.float32),
        pltpu.VMEM((CHUNK,), jnp.int32),
        pltpu.VMEM((HR,), jnp.int32),
        pltpu.SemaphoreType.DMA,
    ],
    compiler_params=pltpu.CompilerParams(needs_layout_passes=False),
)


def _conv(h, mean, wl_ref, wr_ref, bl_ref, br_ref):
    return (jnp.dot(mean, wl_ref[:, :], preferred_element_type=jnp.float32)
            + jnp.dot(h, wr_ref[:, :], preferred_element_type=jnp.float32)
            + bl_ref[:, :] + br_ref[:, :])


def _tc_layer_body(residual, hp_ref, p0_ref, p1_ref, inv_ref, wl_ref, wr_ref,
                   bl_ref, br_ref, o_ref):
    h = hp_ref[:, :]
    mean = (p0_ref[:, :] + p1_ref[:, :]) * inv_ref[:, :]
    z = _conv(h, mean, wl_ref, wr_ref, bl_ref, br_ref)
    if residual:
        z = z + h
        mu = jnp.mean(z, axis=1, keepdims=True)
        var = jnp.mean((z - mu) ** 2, axis=1, keepdims=True)
        z = (z - mu) * lax.rsqrt(var + 1e-5)
    o_ref[:, :] = jnp.maximum(z, 0.0)


def _tc_last_body(hp_ref, p0_ref, p1_ref, inv_ref, wl_ref, wr_ref,
                  bl_ref, br_ref, woutp_ref, boutp_ref, o_ref):
    h = hp_ref[:, :]
    mean = (p0_ref[:, :] + p1_ref[:, :]) * inv_ref[:, :]
    z = _conv(h, mean, wl_ref, wr_ref, bl_ref, br_ref)
    z = z + h
    mu = jnp.mean(z, axis=1, keepdims=True)
    var = jnp.mean((z - mu) ** 2, axis=1, keepdims=True)
    z = (z - mu) * lax.rsqrt(var + 1e-5)
    z = jnp.maximum(z, 0.0)
    logits = jnp.dot(z, woutp_ref[:, :], preferred_element_type=jnp.float32)
    logits = logits + boutp_ref[:, :]
    o_ref[:, :] = logits[:, :2]


_ROW = lambda i: (i, 0)
_FIX = lambda i: (0, 0)


def _tc_layer(residual, h, p0, p1, inv_cnt, Wl, Wr, bl, br):
    return pl.pallas_call(
        functools.partial(_tc_layer_body, residual),
        grid=(N // BR,),
        in_specs=[
            pl.BlockSpec((BR, D), _ROW),
            pl.BlockSpec((BR, D), _ROW),
            pl.BlockSpec((BR, D), _ROW),
            pl.BlockSpec((BR, 1), _ROW),
            pl.BlockSpec((D, D), _FIX),
            pl.BlockSpec((D, D), _FIX),
            pl.BlockSpec((1, D), _FIX),
            pl.BlockSpec((1, D), _FIX),
        ],
        out_specs=pl.BlockSpec((BR, D), _ROW),
        out_shape=jax.ShapeDtypeStruct((N, D), jnp.float32),
    )(h, p0, p1, inv_cnt, Wl, Wr, bl.reshape(1, D), br.reshape(1, D))


def _tc_last(h, p0, p1, inv_cnt, Wl, Wr, bl, br, Woutp, boutp):
    return pl.pallas_call(
        _tc_last_body,
        grid=(N // BR,),
        in_specs=[
            pl.BlockSpec((BR, D), _ROW),
            pl.BlockSpec((BR, D), _ROW),
            pl.BlockSpec((BR, D), _ROW),
            pl.BlockSpec((BR, 1), _ROW),
            pl.BlockSpec((D, D), _FIX),
            pl.BlockSpec((D, D), _FIX),
            pl.BlockSpec((1, D), _FIX),
            pl.BlockSpec((1, D), _FIX),
            pl.BlockSpec((D, D), _FIX),
            pl.BlockSpec((1, D), _FIX),
        ],
        out_specs=pl.BlockSpec((BR, 2), _ROW),
        out_shape=jax.ShapeDtypeStruct((N, 2), jnp.float32),
    )(h, p0, p1, inv_cnt, Wl, Wr, bl.reshape(1, D), br.reshape(1, D),
      Woutp, boutp)


def kernel(x, edge_index, Wl0, bl0, Wr0, br0, Wl1, bl1, Wr1, br1,
           Wl2, bl2, Wr2, br2, Wout, bout):
    f32 = jnp.float32
    x = x.astype(f32)
    src = edge_index[0].astype(jnp.int32)
    dst = edge_index[1].astype(jnp.int32)
    srcp = jnp.concatenate([src, jnp.zeros((EPAD - E,), jnp.int32)])
    dstp = jnp.concatenate([dst, jnp.full((EPAD - E,), N, jnp.int32)])
    zrows = jnp.zeros((RPT, D), f32)
    iota80 = jnp.arange(HR, dtype=jnp.int32)
    Woutp = jnp.zeros((D, D), f32).at[:, :2].set(Wout.astype(f32))
    boutp = jnp.zeros((1, D), f32).at[:, :2].set(bout.astype(f32)[None, :])

    c0, c1 = _sc_degree(dstp, zrows, iota80)
    cnt = (c0 + c1).reshape(NPAD)[:N].reshape(N, 1)
    inv_cnt = 1.0 / jnp.maximum(cnt, 1.0)

    h = x
    p0, p1 = _sc_segsum(h, srcp, dstp, zrows)
    h = _tc_layer(False, h, p0, p1, inv_cnt, Wl0, Wr0, bl0, br0)
    p0, p1 = _sc_segsum(h, srcp, dstp, zrows)
    h = _tc_layer(True, h, p0, p1, inv_cnt, Wl1, Wr1, bl1, br1)
    p0, p1 = _sc_segsum(h, srcp, dstp, zrows)
    return _tc_last(h, p0, p1, inv_cnt, Wl2, Wr2, bl2, br2, Woutp, boutp)


# speedup vs baseline: 3.8616x; 1.2694x over previous
import functools

import jax
import jax.numpy as jnp
from jax import lax
from jax.experimental import pallas as pl
from jax.experimental.pallas import tpu as pltpu
from jax.experimental.pallas import tpu_sc as plsc

N = 10000
E = 320000
D = 128
NC, NS = 2, 16
NT = NC * NS
CHUNK = 128
EPT = 10240
EPAD = NT * EPT
NITER = EPT // CHUNK
NPAD = 10240
RPT = NPAD // NS
OPT = 624
HR = NPAD // D
BR = 1000


NBUF = 2
NHALF = NITER // 2
NGR = NHALF // NBUF


def _sc_segsum_body(h, srcp3, dstp3, zrows, out0, out1,
                    acc, sidx, didx, r0, r1, sg0, sg1):
    bufs = [r0, r1]
    sems = [sg0, sg1]
    c = lax.axis_index("c")
    s = lax.axis_index("s")
    tile = c * NS + s

    pltpu.sync_copy(zrows, acc.at[pl.ds(s * RPT, RPT)])
    plsc.subcore_barrier()

    for half in range(2):
        base = tile * NITER + half * NHALF
        pltpu.sync_copy(srcp3.at[pl.ds(base, NHALF)], sidx)
        pltpu.sync_copy(dstp3.at[pl.ds(base, NHALF)], didx)

        for b in range(NBUF):
            pltpu.async_copy(h.at[sidx.at[b, 0]], bufs[b], sems[b])

        def group(g, carry):
            for b in range(NBUF):
                i = g * NBUF + b
                pltpu.make_async_copy(h.at[sidx.at[i, 0]], bufs[b],
                                      sems[b]).wait()
                pltpu.sync_copy(bufs[b], acc.at[didx.at[i, 0]], add=True)

                @pl.when(g < NGR - 1)
                def _():
                    pltpu.async_copy(h.at[sidx.at[i + NBUF, 0]], bufs[b],
                                     sems[b])
            return carry

        lax.fori_loop(0, NGR, group, 0)
    plsc.subcore_barrier()

    rows = pl.ds(s * OPT, OPT)
    tail = pl.ds(NS * OPT, N - NS * OPT)

    @pl.when(c == 0)
    def _():
        pltpu.sync_copy(acc.at[rows], out0.at[rows])

    @pl.when(c == 1)
    def _():
        pltpu.sync_copy(acc.at[rows], out1.at[rows])

    @pl.when((c == 0) & (s == NS - 1))
    def _():
        pltpu.sync_copy(acc.at[tail], out0.at[tail])

    @pl.when((c == 1) & (s == NS - 1))
    def _():
        pltpu.sync_copy(acc.at[tail], out1.at[tail])


_sc_segsum = pl.kernel(
    _sc_segsum_body,
    out_type=(jax.ShapeDtypeStruct((N, D), jnp.float32),
              jax.ShapeDtypeStruct((N, D), jnp.float32)),
    mesh=plsc.VectorSubcoreMesh(core_axis_name="c", subcore_axis_name="s"),
    scratch_types=(
        [pltpu.VMEM_SHARED((NPAD, D), jnp.float32),
         pltpu.VMEM((NHALF, 1, CHUNK), jnp.int32),
         pltpu.VMEM((NHALF, 1, CHUNK), jnp.int32)]
        + [pltpu.VMEM((CHUNK, D), jnp.float32)] * NBUF
        + [pltpu.SemaphoreType.DMA] * NBUF
    ),
)


def _sc_degree_body(dstp, zrows, iota_hbm, out0, out1,
                    acc, hist, didx_v, idx80, sem):
    c = lax.axis_index("c")
    s = lax.axis_index("s")
    tile = c * NS + s

    @pl.when(s < HR // 8)
    def _():
        pltpu.sync_copy(zrows.at[pl.ds(0, 8)], acc.at[pl.ds(s * 8, 8)])

    pltpu.sync_copy(zrows.at[pl.ds(0, HR)], hist)
    pltpu.sync_copy(iota_hbm, idx80)
    plsc.subcore_barrier()

    def step(i, carry):
        pltpu.sync_copy(dstp.at[tile * NITER + i, 0], didx_v)
        for k in range(CHUNK // 16):
            d16 = didx_v[pl.ds(k * 16, 16)]
            cnt, last = plsc.scan_count(d16)
            plsc.addupdate_scatter(
                hist,
                [lax.shift_right_logical(d16, 7),
                 lax.bitwise_and(d16, 127)],
                cnt.astype(jnp.float32),
                mask=last,
            )
        return carry

    lax.fori_loop(0, NITER, step, 0)
    pltpu.sync_copy(hist, acc.at[idx80], add=True)
    plsc.subcore_barrier()

    @pl.when((c == 0) & (s < HR // 8))
    def _():
        pltpu.sync_copy(acc.at[pl.ds(s * 8, 8)], out0.at[pl.ds(s * 8, 8)])

    @pl.when((c == 1) & (s < HR // 8))
    def _():
        pltpu.sync_copy(acc.at[pl.ds(s * 8, 8)], out1.at[pl.ds(s * 8, 8)])


_sc_degree = pl.kernel(
    _sc_degree_body,
    out_type=(jax.ShapeDtypeStruct((HR, D), jnp.float32),
              jax.ShapeDtypeStruct((HR, D), jnp.float32)),
    mesh=plsc.VectorSubcoreMesh(core_axis_name="c", subcore_axis_name="s"),
    scratch_types=[
        pltpu.VMEM_SHARED((HR, D), jnp.float32),
        pltpu.VMEM((HR, D), jnp.float32),
        pltpu.VMEM((CHUNK,), jnp.int32),
        pltpu.VMEM((HR,), jnp.int32),
        pltpu.SemaphoreType.DMA,
    ],
    compiler_params=pltpu.CompilerParams(needs_layout_passes=False),
)


def _conv(h, mean, wl_ref, wr_ref, bl_ref, br_ref):
    return (jnp.dot(mean, wl_ref[:, :], preferred_element_type=jnp.float32)
            + jnp.dot(h, wr_ref[:, :], preferred_element_type=jnp.float32)
            + bl_ref[:, :] + br_ref[:, :])


def _tc_layer_body(residual, hp_ref, p0_ref, p1_ref, inv_ref, wl_ref, wr_ref,
                   bl_ref, br_ref, o_ref):
    h = hp_ref[:, :]
    mean = (p0_ref[:, :] + p1_ref[:, :]) * inv_ref[:, :]
    z = _conv(h, mean, wl_ref, wr_ref, bl_ref, br_ref)
    if residual:
        z = z + h
        mu = jnp.mean(z, axis=1, keepdims=True)
        var = jnp.mean((z - mu) ** 2, axis=1, keepdims=True)
        z = (z - mu) * lax.rsqrt(var + 1e-5)
    o_ref[:, :] = jnp.maximum(z, 0.0)


def _tc_last_body(hp_ref, p0_ref, p1_ref, inv_ref, wl_ref, wr_ref,
                  bl_ref, br_ref, woutp_ref, boutp_ref, o_ref):
    h = hp_ref[:, :]
    mean = (p0_ref[:, :] + p1_ref[:, :]) * inv_ref[:, :]
    z = _conv(h, mean, wl_ref, wr_ref, bl_ref, br_ref)
    z = z + h
    mu = jnp.mean(z, axis=1, keepdims=True)
    var = jnp.mean((z - mu) ** 2, axis=1, keepdims=True)
    z = (z - mu) * lax.rsqrt(var + 1e-5)
    z = jnp.maximum(z, 0.0)
    logits = jnp.dot(z, woutp_ref[:, :], preferred_element_type=jnp.float32)
    logits = logits + boutp_ref[:, :]
    o_ref[:, :] = logits[:, :2]


_ROW = lambda i: (i, 0)
_FIX = lambda i: (0, 0)


def _tc_layer(residual, h, p0, p1, inv_cnt, Wl, Wr, bl, br):
    return pl.pallas_call(
        functools.partial(_tc_layer_body, residual),
        grid=(N // BR,),
        in_specs=[
            pl.BlockSpec((BR, D), _ROW),
            pl.BlockSpec((BR, D), _ROW),
            pl.BlockSpec((BR, D), _ROW),
            pl.BlockSpec((BR, 1), _ROW),
            pl.BlockSpec((D, D), _FIX),
            pl.BlockSpec((D, D), _FIX),
            pl.BlockSpec((1, D), _FIX),
            pl.BlockSpec((1, D), _FIX),
        ],
        out_specs=pl.BlockSpec((BR, D), _ROW),
        out_shape=jax.ShapeDtypeStruct((N, D), jnp.float32),
    )(h, p0, p1, inv_cnt, Wl, Wr, bl.reshape(1, D), br.reshape(1, D))


def _tc_last(h, p0, p1, inv_cnt, Wl, Wr, bl, br, Woutp, boutp):
    return pl.pallas_call(
        _tc_last_body,
        grid=(N // BR,),
        in_specs=[
            pl.BlockSpec((BR, D), _ROW),
            pl.BlockSpec((BR, D), _ROW),
            pl.BlockSpec((BR, D), _ROW),
            pl.BlockSpec((BR, 1), _ROW),
            pl.BlockSpec((D, D), _FIX),
            pl.BlockSpec((D, D), _FIX),
            pl.BlockSpec((1, D), _FIX),
            pl.BlockSpec((1, D), _FIX),
            pl.BlockSpec((D, D), _FIX),
            pl.BlockSpec((1, D), _FIX),
        ],
        out_specs=pl.BlockSpec((BR, 2), _ROW),
        out_shape=jax.ShapeDtypeStruct((N, 2), jnp.float32),
    )(h, p0, p1, inv_cnt, Wl, Wr, bl.reshape(1, D), br.reshape(1, D),
      Woutp, boutp)


def kernel(x, edge_index, Wl0, bl0, Wr0, br0, Wl1, bl1, Wr1, br1,
           Wl2, bl2, Wr2, br2, Wout, bout):
    f32 = jnp.float32
    x = x.astype(f32)
    src = edge_index[0].astype(jnp.int32)
    dst = edge_index[1].astype(jnp.int32)
    srcp = jnp.concatenate(
        [src, jnp.zeros((EPAD - E,), jnp.int32)]).reshape(-1, 1, CHUNK)
    dstp = jnp.concatenate(
        [dst, jnp.full((EPAD - E,), N, jnp.int32)]).reshape(-1, 1, CHUNK)
    zrows = jnp.zeros((RPT, D), f32)
    iota80 = jnp.arange(HR, dtype=jnp.int32)
    Woutp = jnp.zeros((D, D), f32).at[:, :2].set(Wout.astype(f32))
    boutp = jnp.zeros((1, D), f32).at[:, :2].set(bout.astype(f32)[None, :])

    c0, c1 = _sc_degree(dstp, zrows, iota80)
    cnt = (c0 + c1).reshape(NPAD)[:N].reshape(N, 1)
    inv_cnt = 1.0 / jnp.maximum(cnt, 1.0)

    h = x
    p0, p1 = _sc_segsum(h, srcp, dstp, zrows)
    h = _tc_layer(False, h, p0, p1, inv_cnt, Wl0, Wr0, bl0, br0)
    p0, p1 = _sc_segsum(h, srcp, dstp, zrows)
    h = _tc_layer(True, h, p0, p1, inv_cnt, Wl1, Wr1, bl1, br1)
    p0, p1 = _sc_segsum(h, srcp, dstp, zrows)
    return _tc_last(h, p0, p1, inv_cnt, Wl2, Wr2, bl2, br2, Woutp, boutp)
